# baseline (speedup 1.0000x reference)
import functools

import jax
import jax.numpy as jnp
from jax import lax
from jax.experimental import pallas as pl
from jax.experimental.pallas import tpu as pltpu
from jax.experimental.pallas import tpu_sc as plsc

N = 10000
E = 320000
D_IN = 128
H = 256
HH = H // 2
G = 64

NTILES = 16
NP = 10240
DUMP = N
E_PAD = 327680
EROWS = E_PAD // 128
NCHUNK = EROWS // NTILES
EP16 = E_PAD // NTILES
ROWS16 = NP // NTILES
NSPREAD = 8

_MESH = dict(core_axis_name="c", subcore_axis_name="s",
             num_cores=2, num_subcores=16)


def _hist_body(src_hbm, dst_hbm, hs_out, hd_out, idxv, hb, hr):
    cid = lax.axis_index("c")
    sid = lax.axis_index("s")
    z16 = jnp.zeros((16,), jnp.float32)
    ones16 = jnp.ones((16,), jnp.float32)

    def zbody(k, carry):
        hb[pl.ds(k * 16, 16)] = z16
        return carry

    lax.fori_loop(0, NSPREAD * NP // 16, zbody, 0)

    @pl.when(cid == 0)
    def _():
        pltpu.sync_copy(src_hbm.at[pl.ds(sid * EP16, EP16)], idxv)

    @pl.when(cid == 1)
    def _():
        pltpu.sync_copy(dst_hbm.at[pl.ds(sid * EP16, EP16)], idxv)

    lane = lax.iota(jnp.int32, 16)
    g = lane & (NSPREAD - 1)
    dump = NSPREAD * NP + lane
    mlo = lane < NSPREAD
    mhi = lane >= NSPREAD

    def body(i, carry):
        v = idxv[pl.ds(i * 16, 16)]
        flat = g * NP + v
        f1 = jnp.where(mlo, flat, dump)
        h1 = plsc.load_gather(hb, [f1])
        plsc.store_scatter(hb, [f1], h1 + ones16)
        f2 = jnp.where(mhi, flat, dump)
        h2 = plsc.load_gather(hb, [f2])
        plsc.store_scatter(hb, [f2], h2 + ones16)
        return carry

    lax.fori_loop(0, EP16 // 16, body, 0)

    def rbody(k, carry):
        s = hb[pl.ds(k * 16, 16)]
        for q in range(1, NSPREAD):
            s = s + hb[pl.ds(q * NP + k * 16, 16)]
        hr[pl.ds(k * 16, 16)] = s
        return carry

    lax.fori_loop(0, NP // 16, rbody, 0)

    @pl.when(cid == 0)
    def _():
        pltpu.sync_copy(hr, hs_out.at[sid])

    @pl.when(cid == 1)
    def _():
        pltpu.sync_copy(hr, hd_out.at[sid])


def _make_hist():
    return functools.partial(
        pl.kernel,
        out_type=[
            jax.ShapeDtypeStruct((NTILES, NP), jnp.float32),
            jax.ShapeDtypeStruct((NTILES, NP), jnp.float32),
        ],
        mesh=plsc.VectorSubcoreMesh(**_MESH),
        compiler_params=pltpu.CompilerParams(needs_layout_passes=False),
        scratch_types=[
            pltpu.VMEM((EP16,), jnp.int32),
            pltpu.VMEM((NSPREAD * NP + 16,), jnp.float32),
            pltpu.VMEM((NP,), jnp.float32),
        ],
    )(_hist_body)


PASS_ROWS = NP // 2
ACC_ROWS = PASS_ROWS + 256
PR16 = PASS_ROWS // NTILES
AR16 = ACC_ROWS // NTILES
PACK = 16384


def _agg_body(pidx_hbm, ya_hbm, yb_hbm, oa_hbm, ob_hbm,
              pidx2, pidxc, srcu, didxp, rows0, rows1, zb, acc,
              sem0, sem1, ssem0, ssem1):
    cid = lax.axis_index("c")
    sid = lax.axis_index("s")
    for r in range(16):
        for q in range(HH // 16):
            zb[r, pl.ds(q * 16, 16)] = jnp.zeros((16,), jnp.float32)
    r0 = sid * NCHUNK
    pltpu.sync_copy(pidx_hbm.at[pl.ds(r0, NCHUNK)], pidx2)

    zero16 = jnp.zeros((16,), jnp.int32)
    total = NCHUNK * 128

    def pbody(k, cnts):
        c1, c2 = cnts
        v = pidx2[k // 8, pl.ds((k % 8) * 16, 16)]
        d = v & (PACK - 1)
        m1 = d < PASS_ROWS
        m2 = jnp.logical_not(m1)
        i1 = plsc.cumsum(m1.astype(jnp.int32))
        p1 = c1 + i1 - 1
        plsc.store_scatter(pidxc, [p1 >> 7, p1 & 127], v, mask=m1)
        i2 = plsc.cumsum(m2.astype(jnp.int32))
        p2 = (total - 1) - (c2 + i2 - 1)
        plsc.store_scatter(pidxc, [p2 >> 7, p2 & 127], v, mask=m2)
        c1 = c1 + plsc.all_reduce_population_count(m1)
        c2 = c2 + plsc.all_reduce_population_count(m2)
        return (c1, c2)

    c1, _ = lax.fori_loop(0, NCHUNK * 8, pbody, (zero16, zero16))
    n1 = jnp.max(c1)
    hi1 = (n1 + 127) >> 7
    lo2 = n1 >> 7

    def _gwait(rows, sem):
        pltpu.make_async_copy(ya_hbm.at[srcu.at[0]], rows, sem).wait()

    def _swait(rows, slot, sem):
        pltpu.make_async_copy(rows, acc.at[didxp.at[slot]], sem).wait()

    for p in range(2):
        base = p * PASS_ROWS
        z0 = sid * AR16

        def zbody(k, carry):
            pltpu.sync_copy(zb, acc.at[pl.ds(z0 + k * 16, 16)])
            return carry

        lax.fori_loop(0, AR16 // 16, zbody, 0)

        plsc.subcore_barrier()

        def _gissue(i, slot, rows, sem):
            for q in range(8):
                v = pidxc[i, pl.ds(q * 16, 16)]
                srcu[slot, pl.ds(q * 16, 16)] = v >> 14
                d = v & (PACK - 1)
                dloc = d - base
                inr = (dloc >= 0) & (dloc < PASS_ROWS)
                dv = jnp.where(inr, dloc, PASS_ROWS + (d & 255))
                didxp[slot, pl.ds(q * 16, 16)] = dv

            @pl.when(cid == 0)
            def _():
                pltpu.async_copy(ya_hbm.at[srcu.at[slot]], rows, sem)

            @pl.when(cid == 1)
            def _():
                pltpu.async_copy(yb_hbm.at[srcu.at[slot]], rows, sem)

        lo = 0 if p == 0 else lo2
        hi = hi1 if p == 0 else NCHUNK

        @pl.when(lo < hi)
        def _():
            _gissue(lo, 0, rows0, sem0)

        def body(j, carry):
            i = lo + 2 * j

            @pl.when(j > 0)
            def _():
                _swait(rows1, 1, ssem1)

            @pl.when(i + 1 < hi)
            def _():
                _gissue(i + 1, 1, rows1, sem1)

            _gwait(rows0, sem0)
            pltpu.async_copy(rows0, acc.at[didxp.at[0]], ssem0, add=True)

            @pl.when(i + 1 < hi)
            def _():
                _gwait(rows1, sem1)
                pltpu.async_copy(rows1, acc.at[didxp.at[1]], ssem1, add=True)

            _swait(rows0, 0, ssem0)

            @pl.when(i + 2 < hi)
            def _():
                _gissue(i + 2, 0, rows0, sem0)
            return carry

        nit = (hi - lo + 1) >> 1
        lax.fori_loop(0, nit, body, 0)
        nrange = hi - lo

        @pl.when((nrange >= 2) & ((nrange & 1) == 0))
        def _():
            _swait(rows1, 1, ssem1)

        plsc.subcore_barrier()
        w0 = sid * PR16

        @pl.when(cid == 0)
        def _():
            pltpu.sync_copy(acc.at[pl.ds(w0, PR16)],
                            oa_hbm.at[pl.ds(base + w0, PR16)])

        @pl.when(cid == 1)
        def _():
            pltpu.sync_copy(acc.at[pl.ds(w0, PR16)],
                            ob_hbm.at[pl.ds(base + w0, PR16)])

        if p == 0:
            plsc.subcore_barrier()


def _make_agg():
    return functools.partial(
        pl.kernel,
        out_type=[
            jax.ShapeDtypeStruct((NP, HH), jnp.float32),
            jax.ShapeDtypeStruct((NP, HH), jnp.float32),
        ],
        mesh=plsc.VectorSubcoreMesh(**_MESH),
        compiler_params=pltpu.CompilerParams(needs_layout_passes=False),
        scratch_types=[
            pltpu.VMEM((NCHUNK, 128), jnp.int32),
            pltpu.VMEM((NCHUNK, 128), jnp.int32),
            pltpu.VMEM((2, 128), jnp.int32),
            pltpu.VMEM((2, 128), jnp.int32),
            pltpu.VMEM((128, HH), jnp.float32),
            pltpu.VMEM((128, HH), jnp.float32),
            pltpu.VMEM((16, HH), jnp.float32),
            pltpu.VMEM_SHARED((ACC_ROWS, HH), jnp.float32),
            pltpu.SemaphoreType.DMA,
            pltpu.SemaphoreType.DMA,
            pltpu.SemaphoreType.DMA,
            pltpu.SemaphoreType.DMA,
        ],
    )(_agg_body)


def _prep_body(x_ref, hs_ref, hd_ref, w_ref, ya_ref, yb_ref, a_ref, c_ref):
    a = lax.rsqrt(jnp.sum(hs_ref[...], axis=1, keepdims=True) + 1.0)
    c = lax.rsqrt(jnp.sum(hd_ref[...], axis=1, keepdims=True) + 1.0)
    a_ref[...] = a
    c_ref[...] = c
    y = jnp.dot(a * x_ref[...], w_ref[...], preferred_element_type=jnp.float32)
    ya_ref[...] = y[:, :HH]
    yb_ref[...] = y[:, HH:]


def _mid_body(aa_ref, ab_ref, ya_ref, yb_ref, a_ref, c_ref,
              wa_ref, wb_ref, b_ref, oa_ref, ob_ref):
    c = c_ref[...]
    a = a_ref[...]
    ha = jnp.maximum(c * (aa_ref[...] + ya_ref[...]) + b_ref[:, :HH], 0.0)
    hb = jnp.maximum(c * (ab_ref[...] + yb_ref[...]) + b_ref[:, HH:], 0.0)
    y = jnp.dot(a * ha, wa_ref[...], preferred_element_type=jnp.float32)
    y += jnp.dot(a * hb, wb_ref[...], preferred_element_type=jnp.float32)
    oa_ref[...] = y[:, :HH]
    ob_ref[...] = y[:, HH:]


def _final_body(aa_ref, ab_ref, ya_ref, yb_ref, c_ref, b_ref, n2g_ref,
                nf_ref, gf_ref):
    c = c_ref[...]
    ha = jnp.maximum(c * (aa_ref[...] + ya_ref[...]) + b_ref[:, :HH], 0.0)
    hb = jnp.maximum(c * (ab_ref[...] + yb_ref[...]) + b_ref[:, HH:], 0.0)
    hn = jnp.concatenate([ha[:N, :], hb[:N, :]], axis=1)
    nf_ref[...] = hn
    gids = lax.broadcasted_iota(jnp.int32, (N, G), 1)
    onehot = jnp.where(n2g_ref[...] == gids, 1.0, 0.0).astype(jnp.float32)
    gf_ref[...] = lax.dot_general(
        onehot, hn, (((0,), (0,)), ((), ())),
        preferred_element_type=jnp.float32)


def kernel(input, edge_index, node2graph, positions, W0, b0, W1, b1, W2, b2):
    f32 = jnp.float32
    src = edge_index[0]
    dst = edge_index[1]
    pad = jnp.full((E_PAD - E,), DUMP, dtype=src.dtype)
    src1 = jnp.concatenate([src, pad])
    dst1 = jnp.concatenate([dst, pad])
    pidx2 = (src1 * PACK + dst1).reshape(EROWS, 128)
    xp = jnp.zeros((NP, D_IN), f32).at[:N].set(input)

    hsp, hdp = _make_hist()(src1, dst1)

    y0a, y0b, a2, c2 = pl.pallas_call(
        _prep_body,
        out_shape=[
            jax.ShapeDtypeStruct((NP, HH), f32),
            jax.ShapeDtypeStruct((NP, HH), f32),
            jax.ShapeDtypeStruct((NP, 1), f32),
            jax.ShapeDtypeStruct((NP, 1), f32),
        ],
    )(xp, jnp.swapaxes(hsp, 0, 1), jnp.swapaxes(hdp, 0, 1), W0)

    agg = _make_agg()
    mid_call = pl.pallas_call(
        _mid_body,
        out_shape=[
            jax.ShapeDtypeStruct((NP, HH), f32),
            jax.ShapeDtypeStruct((NP, HH), f32),
        ],
    )

    s0a, s0b = agg(pidx2, y0a, y0b)
    y1a, y1b = mid_call(s0a, s0b, y0a, y0b, a2, c2,
                        W1[:HH], W1[HH:], b0.reshape(1, H))

    s1a, s1b = agg(pidx2, y1a, y1b)
    y2a, y2b = mid_call(s1a, s1b, y1a, y1b, a2, c2,
                        W2[:HH], W2[HH:], b1.reshape(1, H))

    s2a, s2b = agg(pidx2, y2a, y2b)
    node_feature, graph_feature = pl.pallas_call(
        _final_body,
        out_shape=[
            jax.ShapeDtypeStruct((N, H), f32),
            jax.ShapeDtypeStruct((G, H), f32),
        ],
    )(s2a, s2b, y2a, y2b, c2, b2.reshape(1, H), node2graph.reshape(N, 1))

    return (graph_feature, node_feature)


# speedup vs baseline: 1.0784x; 1.0784x over previous
import functools

import jax
import jax.numpy as jnp
from jax import lax
from jax.experimental import pallas as pl
from jax.experimental.pallas import tpu as pltpu
from jax.experimental.pallas import tpu_sc as plsc

N = 10000
E = 320000
D_IN = 128
H = 256
HH = H // 2
G = 64

NTILES = 16
NP = 10240
DUMP = N
E_PAD = 327680
EROWS = E_PAD // 128
NCHUNK = EROWS // NTILES
EP16 = E_PAD // NTILES
ROWS16 = NP // NTILES
NSPREAD = 8

_MESH = dict(core_axis_name="c", subcore_axis_name="s",
             num_cores=2, num_subcores=16)


def _hist_body(src_hbm, dst_hbm, hs_out, hd_out, idxv, hb, hr):
    cid = lax.axis_index("c")
    sid = lax.axis_index("s")
    z16 = jnp.zeros((16,), jnp.float32)
    ones16 = jnp.ones((16,), jnp.float32)

    def zbody(k, carry):
        hb[pl.ds(k * 16, 16)] = z16
        return carry

    lax.fori_loop(0, NSPREAD * NP // 16, zbody, 0)

    @pl.when(cid == 0)
    def _():
        pltpu.sync_copy(src_hbm.at[pl.ds(sid * EP16, EP16)], idxv)

    @pl.when(cid == 1)
    def _():
        pltpu.sync_copy(dst_hbm.at[pl.ds(sid * EP16, EP16)], idxv)

    lane = lax.iota(jnp.int32, 16)
    g = lane & (NSPREAD - 1)
    dump = NSPREAD * NP + lane
    mlo = lane < NSPREAD
    mhi = lane >= NSPREAD

    def body(i, carry):
        v = idxv[pl.ds(i * 16, 16)]
        flat = g * NP + v
        f1 = jnp.where(mlo, flat, dump)
        h1 = plsc.load_gather(hb, [f1])
        plsc.store_scatter(hb, [f1], h1 + ones16)
        f2 = jnp.where(mhi, flat, dump)
        h2 = plsc.load_gather(hb, [f2])
        plsc.store_scatter(hb, [f2], h2 + ones16)
        return carry

    lax.fori_loop(0, EP16 // 16, body, 0)

    def rbody(k, carry):
        s = hb[pl.ds(k * 16, 16)]
        for q in range(1, NSPREAD):
            s = s + hb[pl.ds(q * NP + k * 16, 16)]
        hr[pl.ds(k * 16, 16)] = s
        return carry

    lax.fori_loop(0, NP // 16, rbody, 0)

    @pl.when(cid == 0)
    def _():
        pltpu.sync_copy(hr, hs_out.at[sid])

    @pl.when(cid == 1)
    def _():
        pltpu.sync_copy(hr, hd_out.at[sid])


def _make_hist():
    return functools.partial(
        pl.kernel,
        out_type=[
            jax.ShapeDtypeStruct((NTILES, NP), jnp.float32),
            jax.ShapeDtypeStruct((NTILES, NP), jnp.float32),
        ],
        mesh=plsc.VectorSubcoreMesh(**_MESH),
        compiler_params=pltpu.CompilerParams(needs_layout_passes=False),
        scratch_types=[
            pltpu.VMEM((EP16,), jnp.int32),
            pltpu.VMEM((NSPREAD * NP + 16,), jnp.float32),
            pltpu.VMEM((NP,), jnp.float32),
        ],
    )(_hist_body)


PASS_ROWS = NP // 2
ACC_ROWS = PASS_ROWS + 256
PR16 = PASS_ROWS // NTILES
AR16 = ACC_ROWS // NTILES
PACK = 16384


def _agg_body(pidx_hbm, ya_hbm, yb_hbm, oa_hbm, ob_hbm,
              pidx2, pidxc, srcu, didxp, rows0, rows1, zb, acc, sem0, sem1):
    cid = lax.axis_index("c")
    sid = lax.axis_index("s")
    for r in range(16):
        for q in range(HH // 16):
            zb[r, pl.ds(q * 16, 16)] = jnp.zeros((16,), jnp.float32)
    r0 = sid * NCHUNK
    pltpu.sync_copy(pidx_hbm.at[pl.ds(r0, NCHUNK)], pidx2)

    zero16 = jnp.zeros((16,), jnp.int32)
    total = NCHUNK * 128

    def pbody(k, cnts):
        c1, c2 = cnts
        v = pidx2[k // 8, pl.ds((k % 8) * 16, 16)]
        d = v & (PACK - 1)
        m1 = d < PASS_ROWS
        m2 = jnp.logical_not(m1)
        i1 = plsc.cumsum(m1.astype(jnp.int32))
        p1 = c1 + i1 - 1
        plsc.store_scatter(pidxc, [p1 >> 7, p1 & 127], v, mask=m1)
        i2 = plsc.cumsum(m2.astype(jnp.int32))
        p2 = (total - 1) - (c2 + i2 - 1)
        plsc.store_scatter(pidxc, [p2 >> 7, p2 & 127], v, mask=m2)
        c1 = c1 + plsc.all_reduce_population_count(m1)
        c2 = c2 + plsc.all_reduce_population_count(m2)
        return (c1, c2)

    c1, _ = lax.fori_loop(0, NCHUNK * 8, pbody, (zero16, zero16))
    n1 = jnp.max(c1)
    hi1 = (n1 + 127) >> 7
    lo2 = n1 >> 7

    def _gwait(rows, sem):
        pltpu.make_async_copy(ya_hbm.at[srcu.at[0]], rows, sem).wait()

    for p in range(2):
        base = p * PASS_ROWS
        z0 = sid * AR16

        def zbody(k, carry):
            pltpu.sync_copy(zb, acc.at[pl.ds(z0 + k * 16, 16)])
            return carry

        lax.fori_loop(0, AR16 // 16, zbody, 0)

        plsc.subcore_barrier()

        def _gissue(i, slot, rows, sem):
            for q in range(8):
                v = pidxc[i, pl.ds(q * 16, 16)]
                srcu[slot, pl.ds(q * 16, 16)] = v >> 14
                d = v & (PACK - 1)
                dloc = d - base
                inr = (dloc >= 0) & (dloc < PASS_ROWS)
                dv = jnp.where(inr, dloc, PASS_ROWS + (d & 255))
                didxp[slot, pl.ds(q * 16, 16)] = dv

            @pl.when(cid == 0)
            def _():
                pltpu.async_copy(ya_hbm.at[srcu.at[slot]], rows, sem)

            @pl.when(cid == 1)
            def _():
                pltpu.async_copy(yb_hbm.at[srcu.at[slot]], rows, sem)

        lo = 0 if p == 0 else lo2
        hi = hi1 if p == 0 else NCHUNK

        @pl.when(lo < hi)
        def _():
            _gissue(lo, 0, rows0, sem0)

        def body(j, carry):
            i = lo + 2 * j
            @pl.when(i + 1 < hi)
            def _():
                _gissue(i + 1, 1, rows1, sem1)

            _gwait(rows0, sem0)
            pltpu.sync_copy(rows0, acc.at[didxp.at[0]], add=True)

            @pl.when(i + 2 < hi)
            def _():
                _gissue(i + 2, 0, rows0, sem0)

            @pl.when(i + 1 < hi)
            def _():
                _gwait(rows1, sem1)
                pltpu.sync_copy(rows1, acc.at[didxp.at[1]], add=True)
            return carry

        lax.fori_loop(0, (hi - lo + 1) >> 1, body, 0)
        plsc.subcore_barrier()
        w0 = sid * PR16

        @pl.when(cid == 0)
        def _():
            pltpu.sync_copy(acc.at[pl.ds(w0, PR16)],
                            oa_hbm.at[pl.ds(base + w0, PR16)])

        @pl.when(cid == 1)
        def _():
            pltpu.sync_copy(acc.at[pl.ds(w0, PR16)],
                            ob_hbm.at[pl.ds(base + w0, PR16)])

        if p == 0:
            plsc.subcore_barrier()


def _make_agg():
    return functools.partial(
        pl.kernel,
        out_type=[
            jax.ShapeDtypeStruct((NP, HH), jnp.float32),
            jax.ShapeDtypeStruct((NP, HH), jnp.float32),
        ],
        mesh=plsc.VectorSubcoreMesh(**_MESH),
        compiler_params=pltpu.CompilerParams(needs_layout_passes=False),
        scratch_types=[
            pltpu.VMEM((NCHUNK, 128), jnp.int32),
            pltpu.VMEM((NCHUNK, 128), jnp.int32),
            pltpu.VMEM((2, 128), jnp.int32),
            pltpu.VMEM((2, 128), jnp.int32),
            pltpu.VMEM((128, HH), jnp.float32),
            pltpu.VMEM((128, HH), jnp.float32),
            pltpu.VMEM((16, HH), jnp.float32),
            pltpu.VMEM_SHARED((ACC_ROWS, HH), jnp.float32),
            pltpu.SemaphoreType.DMA,
            pltpu.SemaphoreType.DMA,
        ],
    )(_agg_body)


def _prep_body(x_ref, hs_ref, hd_ref, w_ref, ya_ref, yb_ref, a_ref, c_ref):
    a = lax.rsqrt(jnp.sum(hs_ref[...], axis=1, keepdims=True) + 1.0)
    c = lax.rsqrt(jnp.sum(hd_ref[...], axis=1, keepdims=True) + 1.0)
    a_ref[...] = a
    c_ref[...] = c
    y = jnp.dot(a * x_ref[...], w_ref[...], preferred_element_type=jnp.float32)
    ya_ref[...] = y[:, :HH]
    yb_ref[...] = y[:, HH:]


def _mid_body(aa_ref, ab_ref, ya_ref, yb_ref, a_ref, c_ref,
              wa_ref, wb_ref, b_ref, oa_ref, ob_ref):
    c = c_ref[...]
    a = a_ref[...]
    ha = jnp.maximum(c * (aa_ref[...] + ya_ref[...]) + b_ref[:, :HH], 0.0)
    hb = jnp.maximum(c * (ab_ref[...] + yb_ref[...]) + b_ref[:, HH:], 0.0)
    y = jnp.dot(a * ha, wa_ref[...], preferred_element_type=jnp.float32)
    y += jnp.dot(a * hb, wb_ref[...], preferred_element_type=jnp.float32)
    oa_ref[...] = y[:, :HH]
    ob_ref[...] = y[:, HH:]


def _final_body(aa_ref, ab_ref, ya_ref, yb_ref, c_ref, b_ref, n2g_ref,
                nf_ref, gf_ref):
    c = c_ref[...]
    ha = jnp.maximum(c * (aa_ref[...] + ya_ref[...]) + b_ref[:, :HH], 0.0)
    hb = jnp.maximum(c * (ab_ref[...] + yb_ref[...]) + b_ref[:, HH:], 0.0)
    hn = jnp.concatenate([ha[:N, :], hb[:N, :]], axis=1)
    nf_ref[...] = hn
    gids = lax.broadcasted_iota(jnp.int32, (N, G), 1)
    onehot = jnp.where(n2g_ref[...] == gids, 1.0, 0.0).astype(jnp.float32)
    gf_ref[...] = lax.dot_general(
        onehot, hn, (((0,), (0,)), ((), ())),
        preferred_element_type=jnp.float32)


def kernel(input, edge_index, node2graph, positions, W0, b0, W1, b1, W2, b2):
    f32 = jnp.float32
    src = edge_index[0]
    dst = edge_index[1]
    pad = jnp.full((E_PAD - E,), DUMP, dtype=src.dtype)
    src1 = jnp.concatenate([src, pad])
    dst1 = jnp.concatenate([dst, pad])
    pidx2 = (src1 * PACK + dst1).reshape(EROWS, 128)
    xp = jnp.zeros((NP, D_IN), f32).at[:N].set(input)

    hsp, hdp = _make_hist()(src1, dst1)

    y0a, y0b, a2, c2 = pl.pallas_call(
        _prep_body,
        out_shape=[
            jax.ShapeDtypeStruct((NP, HH), f32),
            jax.ShapeDtypeStruct((NP, HH), f32),
            jax.ShapeDtypeStruct((NP, 1), f32),
            jax.ShapeDtypeStruct((NP, 1), f32),
        ],
    )(xp, jnp.swapaxes(hsp, 0, 1), jnp.swapaxes(hdp, 0, 1), W0)

    agg = _make_agg()
    mid_call = pl.pallas_call(
        _mid_body,
        out_shape=[
            jax.ShapeDtypeStruct((NP, HH), f32),
            jax.ShapeDtypeStruct((NP, HH), f32),
        ],
    )

    s0a, s0b = agg(pidx2, y0a, y0b)
    y1a, y1b = mid_call(s0a, s0b, y0a, y0b, a2, c2,
                        W1[:HH], W1[HH:], b0.reshape(1, H))

    s1a, s1b = agg(pidx2, y1a, y1b)
    y2a, y2b = mid_call(s1a, s1b, y1a, y1b, a2, c2,
                        W2[:HH], W2[HH:], b1.reshape(1, H))

    s2a, s2b = agg(pidx2, y2a, y2b)
    node_feature, graph_feature = pl.pallas_call(
        _final_body,
        out_shape=[
            jax.ShapeDtypeStruct((N, H), f32),
            jax.ShapeDtypeStruct((G, H), f32),
        ],
    )(s2a, s2b, y2a, y2b, c2, b2.reshape(1, H), node2graph.reshape(N, 1))

    return (graph_feature, node_feature)
